# Initial kernel scaffold; baseline (speedup 1.0000x reference)
import jax
import jax.numpy as jnp
from jax.experimental import pallas as pl


def kernel(seq, table, w1, b1, w2, b2, ln_g, ln_b, wg, bg, wr, br, wo, bo):
    raise NotImplementedError("write your pallas kernel here")


# speedup vs baseline: 1.8985x; 1.8985x over previous
import functools

import jax
import jax.numpy as jnp
from jax import lax
from jax.experimental import pallas as pl
from jax.experimental.pallas import tpu as pltpu
from jax.experimental.pallas import tpu_sc as plsc

B = 1024
T = 200
V = 100000
D = 64
S = 128
KW = 4


_CHUNK = 128
_NBUF = 10


def _sc_gather(table, idx3d):
    info = plsc.get_sparse_core_info()
    nw = info.num_cores * info.num_subcores
    chunks_per_w = idx3d.shape[1]
    n_rows = nw * chunks_per_w * _CHUNK
    groups = chunks_per_w // _NBUF

    mesh = plsc.VectorSubcoreMesh(core_axis_name="c", subcore_axis_name="s")

    @functools.partial(
        pl.kernel,
        mesh=mesh,
        out_type=jax.ShapeDtypeStruct((n_rows, D), jnp.float32),
        scratch_types=[
            pltpu.VMEM((chunks_per_w, _CHUNK), jnp.int32),
            pltpu.VMEM((_NBUF, _CHUNK, D), jnp.float32),
            pltpu.SemaphoreType.DMA,
            pltpu.SemaphoreType.DMA,
        ],
        compiler_params=pltpu.CompilerParams(use_tc_tiling_on_sc=False),
    )
    def k(table_hbm, idx_hbm, out_hbm, idx_v, rows_v, gsem, osem):
        wid = lax.axis_index("s") * info.num_cores + lax.axis_index("c")
        chunk0 = wid * chunks_per_w
        pltpu.sync_copy(idx_hbm.at[wid], idx_v)

        def group(g, _):
            base = g * _NBUF
            for b in range(_NBUF):
                pltpu.async_copy(
                    table_hbm.at[idx_v.at[base + b]], rows_v.at[b], gsem)
            for b in range(_NBUF):
                pltpu.make_async_copy(
                    table_hbm.at[idx_v.at[base + b]], rows_v.at[b], gsem).wait()
            for b in range(_NBUF):
                row0 = (chunk0 + base + b) * _CHUNK
                pltpu.async_copy(
                    rows_v.at[b], out_hbm.at[pl.ds(row0, _CHUNK)], osem)
            for b in range(_NBUF):
                row0 = (chunk0 + base + b) * _CHUNK
                pltpu.make_async_copy(
                    rows_v.at[b], out_hbm.at[pl.ds(row0, _CHUNK)], osem).wait()
            return ()

        lax.fori_loop(0, groups, group, (), unroll=False)

    return k(table, idx3d)


_BT = 64


def _encoder_body(h_ref, w1_ref, b1_ref, w2_ref, b2_ref, lng_ref, lnb_ref,
                  wg_ref, bg_ref, wr_ref, br_ref, ctx_ref):
    h = h_ref[...]
    x = h.reshape(_BT * T, D)
    ff = jnp.maximum(
        jnp.dot(x, w1_ref[...], preferred_element_type=jnp.float32)
        + b1_ref[...], 0.0)
    ff = jnp.dot(ff, w2_ref[...], preferred_element_type=jnp.float32) \
        + b2_ref[...]
    y = x + ff
    m = y.mean(axis=-1, keepdims=True)
    v = ((y - m) ** 2).mean(axis=-1, keepdims=True)
    hid = (y - m) / jnp.sqrt(v + 1e-5) * lng_ref[...] + lnb_ref[...]

    gate = jnp.dot(hid, wg_ref[...], preferred_element_type=jnp.float32) \
        + bg_ref[...]
    scores = gate.mean(axis=-1).reshape(_BT, T)
    tpos = lax.broadcasted_iota(jnp.int32, (_BT, T), 1)
    neg = jnp.float32(-1e30)
    scores = jnp.where(tpos >= T - 1, neg, scores)

    hid3 = hid.reshape(_BT, T, D)
    q = jnp.dot(hid3[:, T - 1, :], wr_ref[...],
                preferred_element_type=jnp.float32) + br_ref[...]
    d_all = (hid3 * q[:, None, :]).sum(axis=-1)

    sel_logit = []
    sel_mask = []
    work = scores
    big = jnp.int32(2 * T)
    for _ in range(KW):
        mx = work.max(axis=1, keepdims=True)
        cand = jnp.where(work == mx, tpos, big)
        pick = cand.min(axis=1, keepdims=True)
        onehot = tpos == pick
        sel_mask.append(onehot)
        sel_logit.append(jnp.where(onehot, d_all, 0.0).sum(axis=1))
        work = jnp.where(onehot, neg, work)

    s = jnp.stack(sel_logit, axis=1)
    mmax = jnp.maximum(s.max(axis=1), 0.0)
    e = jnp.exp(s - mmax[:, None])
    z = e.sum(axis=1) + (S - KW) * jnp.exp(-mmax)
    a = e / z[:, None]

    w_t = jnp.zeros((_BT, T), jnp.float32)
    for kk in range(KW):
        w_t = w_t + jnp.where(sel_mask[kk], a[:, kk:kk + 1], 0.0)
    ctx_ref[...] = (w_t[:, :, None] * hid3).sum(axis=1)


def _encoder(h, w1, b1, w2, b2, ln_g, ln_b, wg, bg, wr, br):
    grid = B // _BT
    full = lambda shape: pl.BlockSpec(shape, lambda i: (0,) * len(shape))
    return pl.pallas_call(
        _encoder_body,
        grid=(grid,),
        in_specs=[
            pl.BlockSpec((_BT, T, D), lambda i: (i, 0, 0)),
            full((D, 2 * D)), full((2 * D,)),
            full((2 * D, D)), full((D,)),
            full((D,)), full((D,)),
            full((D, S)), full((S,)),
            full((D, D)), full((D,)),
        ],
        out_specs=pl.BlockSpec((_BT, D), lambda i: (i, 0)),
        out_shape=jax.ShapeDtypeStruct((B, D), jnp.float32),
    )(h, w1, b1, w2, b2, ln_g, ln_b, wg, bg, wr, br)


_VT = 2048


def _proj_body(ctx_ref, wo_ref, bo_ref, out_ref):
    out_ref[...] = jnp.dot(ctx_ref[...], wo_ref[...],
                           preferred_element_type=jnp.float32) + bo_ref[...]


def _projection(ctx, wo, bo2d):
    grid = pl.cdiv(V, _VT)
    return pl.pallas_call(
        _proj_body,
        grid=(grid,),
        in_specs=[
            pl.BlockSpec((B, D), lambda j: (0, 0)),
            pl.BlockSpec((D, _VT), lambda j: (0, j)),
            pl.BlockSpec((1, _VT), lambda j: (0, j)),
        ],
        out_specs=pl.BlockSpec((B, _VT), lambda j: (0, j)),
        out_shape=jax.ShapeDtypeStruct((B, V), jnp.float32),
    )(ctx, wo, bo2d)


def kernel(seq, table, w1, b1, w2, b2, ln_g, ln_b, wg, bg, wr, br, wo, bo):
    idx3d = seq.reshape(32, -1, _CHUNK).astype(jnp.int32)
    h = _sc_gather(table, idx3d)
    ctx = _encoder(h.reshape(B, T, D), w1, b1, w2, b2,
                   ln_g, ln_b, wg, bg, wr, br)
    return _projection(ctx, wo, bo.reshape(1, V))
